# baseline (speedup 1.0000x reference)
import jax
import jax.numpy as jnp
from jax import lax
from jax.experimental import pallas as pl
from jax.experimental.pallas import tpu as pltpu
from jax.experimental.pallas import tpu_sc as plsc

_S = 50
_B = 4096
_D = 300
_SP = 64
_NC = 2
_NS = 16
_NW = _NC * _NS
_COLS = _B // _NW
_PD = 1024


def _sc_pool_body(table_hbm, xt_hbm, xg_hbm, out_hbm,
                  idx_a, idx_b, gidx_a, gidx_b, rows_a, rows_b,
                  staging, sem_a, sem_b):
    wid = lax.axis_index("s") * _NC + lax.axis_index("c")
    base = wid * _COLS
    zeros16 = jnp.zeros((16,), jnp.float32)
    ones16 = jnp.full((16,), 1.0, jnp.float32)

    for k in range(8):
        staging[pl.ds(896 + 16 * k, 16)] = zeros16

    def fetch(col, idx, gidx, rows, sem):
        pltpu.sync_copy(xt_hbm.at[col], idx)
        pltpu.sync_copy(xg_hbm.at[col], gidx)
        pltpu.make_async_copy(table_hbm.at[gidx], rows, sem).start()

    def compute(col, idx, gidx, rows, sem):
        pltpu.make_async_copy(table_hbm.at[gidx], rows, sem).wait()
        cnt = jnp.zeros((16,), jnp.float32)
        for k in range(4):
            cnt = cnt + jnp.where(idx[pl.ds(16 * k, 16)] != 1,
                                  ones16, zeros16)
        inv = ones16 / jnp.full((16,), jnp.sum(cnt))
        for j in range(19):
            off = 284 if j == 18 else 16 * j

            def body(i, carry, off=off):
                a_s, a_mx, a_mn = carry
                for u in range(10):
                    v = rows[i * 10 + u, pl.ds(off, 16)]
                    a_s = a_s + v
                    a_mx = jnp.maximum(a_mx, v)
                    a_mn = jnp.minimum(a_mn, v)
                return a_s, a_mx, a_mn

            init = (zeros16,
                    jnp.full((16,), -jnp.inf, jnp.float32),
                    jnp.full((16,), jnp.inf, jnp.float32))
            a_s, a_mx, a_mn = lax.fori_loop(0, _S // 10, body, init)
            staging[pl.ds(off, 16)] = a_mx
            staging[pl.ds(300 + off, 16)] = a_s * inv
            staging[pl.ds(600 + off, 16)] = a_mn
        pltpu.sync_copy(staging, out_hbm.at[col])

    fetch(base, idx_a, gidx_a, rows_a, sem_a)

    def loop_body(it, carry):
        col0 = base + 2 * it
        fetch(col0 + 1, idx_b, gidx_b, rows_b, sem_b)
        compute(col0, idx_a, gidx_a, rows_a, sem_a)

        @pl.when(it < _COLS // 2 - 1)
        def _():
            fetch(col0 + 2, idx_a, gidx_a, rows_a, sem_a)

        compute(col0 + 1, idx_b, gidx_b, rows_b, sem_b)
        return carry

    lax.fori_loop(0, _COLS // 2, loop_body, 0)


def _sc_pool(table, xtp, xg):
    mesh = plsc.VectorSubcoreMesh(core_axis_name="c", subcore_axis_name="s")
    f = pl.kernel(
        _sc_pool_body,
        out_type=jax.ShapeDtypeStruct((_B, _PD), jnp.float32),
        mesh=mesh,
        compiler_params=pltpu.CompilerParams(use_tc_tiling_on_sc=False,
                                             needs_layout_passes=False),
        scratch_types=[
            pltpu.VMEM((_SP,), jnp.int32),
            pltpu.VMEM((_SP,), jnp.int32),
            pltpu.VMEM((_S,), jnp.int32),
            pltpu.VMEM((_S,), jnp.int32),
            pltpu.VMEM((_S, _D), jnp.float32),
            pltpu.VMEM((_S, _D), jnp.float32),
            pltpu.VMEM((_PD,), jnp.float32),
            pltpu.SemaphoreType.DMA,
            pltpu.SemaphoreType.DMA,
        ],
    )
    return f(table, xtp, xg)


def _tc_head_body(p_ref, ag_ref, w1_ref, w2_ref, b_ref, o_ref):
    acc = jnp.dot(p_ref[...], w1_ref[...], preferred_element_type=jnp.float32)
    acc = acc + jnp.dot(ag_ref[...], w2_ref[...],
                        preferred_element_type=jnp.float32)
    acc = acc + b_ref[...]
    cols = lax.broadcasted_iota(jnp.int32, acc.shape, 1)
    acc = jnp.where(cols < 10, acc, -jnp.inf)
    m = jnp.max(acc, axis=1, keepdims=True)
    lse = jnp.log(jnp.sum(jnp.exp(acc - m), axis=1, keepdims=True)) + m
    o_ref[...] = acc - lse


def _tc_head(pooled, ag, w1, w2, bp):
    return pl.pallas_call(
        _tc_head_body,
        grid=(16,),
        in_specs=[
            pl.BlockSpec((_B // 16, _PD), lambda i: (i, 0)),
            pl.BlockSpec((_B // 16, 128), lambda i: (i, 0)),
            pl.BlockSpec((_PD, 128), lambda i: (0, 0)),
            pl.BlockSpec((128, 128), lambda i: (0, 0)),
            pl.BlockSpec((1, 128), lambda i: (0, 0)),
        ],
        out_specs=pl.BlockSpec((_B // 16, 128), lambda i: (i, 0)),
        out_shape=jax.ShapeDtypeStruct((_B, 128), jnp.float32),
    )(pooled, ag, w1, w2, bp)


def kernel(x, age, gender, table, W, b):
    xg = x.T
    xtp = jnp.full((_B, _SP), 1, jnp.int32).at[:, :_S].set(xg)
    pooled = _sc_pool(table, xtp, xg)
    ag = (jnp.zeros((_B, 128), jnp.float32)
          .at[:, :11].set(age).at[:, 11:13].set(gender))
    w1 = jnp.zeros((_PD, 128), jnp.float32).at[:900, :10].set(W[:, :900].T)
    w2 = jnp.zeros((128, 128), jnp.float32).at[:13, :10].set(W[:, 900:].T)
    bp = jnp.zeros((1, 128), jnp.float32).at[0, :10].set(b)
    out = _tc_head(pooled, ag, w1, w2, bp)
    return out[:, :10]


# speedup vs baseline: 1.0947x; 1.0947x over previous
import jax
import jax.numpy as jnp
from jax import lax
from jax.experimental import pallas as pl
from jax.experimental.pallas import tpu as pltpu
from jax.experimental.pallas import tpu_sc as plsc

_S = 50
_B = 4096
_D = 300
_SP = 64
_NC = 2
_NS = 16
_NW = _NC * _NS
_COLS = _B // _NW
_PD = 1024


def _sc_pool_body(table_hbm, xt_hbm, xg_hbm, out_hbm,
                  xt_blk, xg_blk, rows_a, rows_b, out_blk,
                  sem_a, sem_b):
    wid = lax.axis_index("s") * _NC + lax.axis_index("c")
    base = wid * _COLS
    zeros16 = jnp.zeros((16,), jnp.float32)
    ones16 = jnp.full((16,), 1.0, jnp.float32)

    pltpu.sync_copy(xt_hbm.at[pl.ds(base, _COLS)], xt_blk)
    pltpu.sync_copy(xg_hbm.at[pl.ds(base, _COLS)], xg_blk)
    for r in range(16):
        for k in range(8):
            out_blk[r, pl.ds(896 + 16 * k, 16)] = zeros16

    def fetch(c, rows, sem):
        pltpu.make_async_copy(table_hbm.at[xg_blk.at[c]], rows, sem).start()

    def compute(c, rows, sem):
        pltpu.make_async_copy(table_hbm.at[xg_blk.at[c]], rows, sem).wait()
        r = lax.rem(c, 16)
        cnt = jnp.zeros((16,), jnp.float32)
        for k in range(4):
            cnt = cnt + jnp.where(xt_blk[c, pl.ds(16 * k, 16)] != 1,
                                  ones16, zeros16)
        inv = ones16 / jnp.full((16,), jnp.sum(cnt))
        for j in range(19):
            off = 284 if j == 18 else 16 * j

            def body(i, carry, off=off):
                a_s, a_mx, a_mn = carry
                for u in range(10):
                    v = rows[i * 10 + u, pl.ds(off, 16)]
                    a_s = a_s + v
                    a_mx = jnp.maximum(a_mx, v)
                    a_mn = jnp.minimum(a_mn, v)
                return a_s, a_mx, a_mn

            init = (zeros16,
                    jnp.full((16,), -jnp.inf, jnp.float32),
                    jnp.full((16,), jnp.inf, jnp.float32))
            a_s, a_mx, a_mn = lax.fori_loop(0, _S // 10, body, init)
            out_blk[r, pl.ds(off, 16)] = a_mx
            out_blk[r, pl.ds(300 + off, 16)] = a_s * inv
            out_blk[r, pl.ds(600 + off, 16)] = a_mn

    fetch(0, rows_a, sem_a)

    def loop_body(it, carry):
        c0 = 2 * it
        fetch(c0 + 1, rows_b, sem_b)
        compute(c0, rows_a, sem_a)

        @pl.when(it < _COLS // 2 - 1)
        def _():
            fetch(c0 + 2, rows_a, sem_a)

        compute(c0 + 1, rows_b, sem_b)

        @pl.when(lax.rem(it, 8) == 7)
        def _():
            grp = lax.div(it, 8)
            pltpu.sync_copy(out_blk, out_hbm.at[pl.ds(base + grp * 16, 16)])

        return carry

    lax.fori_loop(0, _COLS // 2, loop_body, 0)


def _sc_pool(table, xtp, xg):
    mesh = plsc.VectorSubcoreMesh(core_axis_name="c", subcore_axis_name="s")
    f = pl.kernel(
        _sc_pool_body,
        out_type=jax.ShapeDtypeStruct((_B, _PD), jnp.float32),
        mesh=mesh,
        compiler_params=pltpu.CompilerParams(use_tc_tiling_on_sc=False,
                                             needs_layout_passes=False),
        scratch_types=[
            pltpu.VMEM((_COLS, _SP), jnp.int32),
            pltpu.VMEM((_COLS, _S), jnp.int32),
            pltpu.VMEM((_S, _D), jnp.float32),
            pltpu.VMEM((_S, _D), jnp.float32),
            pltpu.VMEM((16, _PD), jnp.float32),
            pltpu.SemaphoreType.DMA,
            pltpu.SemaphoreType.DMA,
        ],
    )
    return f(table, xtp, xg)


def _tc_head_body(p_ref, ag_ref, w1_ref, w2_ref, b_ref, o_ref):
    acc = jnp.dot(p_ref[...], w1_ref[...], preferred_element_type=jnp.float32)
    acc = acc + jnp.dot(ag_ref[...], w2_ref[...],
                        preferred_element_type=jnp.float32)
    acc = acc + b_ref[...]
    cols = lax.broadcasted_iota(jnp.int32, acc.shape, 1)
    acc = jnp.where(cols < 10, acc, -jnp.inf)
    m = jnp.max(acc, axis=1, keepdims=True)
    lse = jnp.log(jnp.sum(jnp.exp(acc - m), axis=1, keepdims=True)) + m
    o_ref[...] = acc - lse


def _tc_head(pooled, ag, w1, w2, bp):
    return pl.pallas_call(
        _tc_head_body,
        grid=(16,),
        in_specs=[
            pl.BlockSpec((_B // 16, _PD), lambda i: (i, 0)),
            pl.BlockSpec((_B // 16, 128), lambda i: (i, 0)),
            pl.BlockSpec((_PD, 128), lambda i: (0, 0)),
            pl.BlockSpec((128, 128), lambda i: (0, 0)),
            pl.BlockSpec((1, 128), lambda i: (0, 0)),
        ],
        out_specs=pl.BlockSpec((_B // 16, 128), lambda i: (i, 0)),
        out_shape=jax.ShapeDtypeStruct((_B, 128), jnp.float32),
    )(pooled, ag, w1, w2, bp)


def kernel(x, age, gender, table, W, b):
    xg = x.T
    xtp = jnp.full((_B, _SP), 1, jnp.int32).at[:, :_S].set(xg)
    table2 = jnp.pad(table, ((0, 8), (0, 0)))
    pooled = _sc_pool(table2, xtp, xg)
    ag = (jnp.zeros((_B, 128), jnp.float32)
          .at[:, :11].set(age).at[:, 11:13].set(gender))
    w1 = jnp.zeros((_PD, 128), jnp.float32).at[:900, :10].set(W[:, :900].T)
    w2 = jnp.zeros((128, 128), jnp.float32).at[:13, :10].set(W[:, 900:].T)
    bp = jnp.zeros((1, 128), jnp.float32).at[0, :10].set(b)
    out = _tc_head(pooled, ag, w1, w2, bp)
    return out[:, :10]
